# baseline (speedup 1.0000x reference)
import functools

import jax
import jax.numpy as jnp
from jax import lax
from jax.experimental import pallas as pl
from jax.experimental.pallas import tpu as pltpu
from jax.experimental.pallas import tpu_sc as plsc

_B = 16384
_D = 16
_NW = 32
_BPW = _B // _NW
_CHUNK = 128
_NCHUNK = _BPW // _CHUNK
_NGRP = _BPW // 16
_REG = 0.01


@functools.partial(
    pl.kernel,
    mesh=plsc.VectorSubcoreMesh(core_axis_name="c", subcore_axis_name="s"),
    compiler_params=pltpu.CompilerParams(
        needs_layout_passes=False, use_tc_tiling_on_sc=False),
    out_type=[
        jax.ShapeDtypeStruct((_B,), jnp.float32),
        jax.ShapeDtypeStruct((2, _NW, _D), jnp.float32),
    ],
    scratch_types=[
        pltpu.VMEM((_BPW,), jnp.int32),
        pltpu.VMEM((_BPW,), jnp.int32),
        pltpu.VMEM((_BPW, _D), jnp.float32),
        pltpu.VMEM((_BPW, _D), jnp.float32),
        pltpu.VMEM((_D,), jnp.float32),
        pltpu.VMEM((_BPW,), jnp.float32),
        pltpu.VMEM((_D,), jnp.float32),
        pltpu.VMEM((_D,), jnp.float32),
        pltpu.SemaphoreType.DMA,
    ],
)
def _gmf_sc(users_hbm, items_hbm, u_emb_hbm, i_emb_hbm, w_hbm,
            out_hbm, parts_hbm,
            idx_u, idx_i, u_rows, i_rows, w_v, out_v, au_v, ai_v, sem):
    wid = lax.axis_index("s") * 2 + lax.axis_index("c")
    base = wid * _BPW

    pltpu.sync_copy(users_hbm.at[pl.ds(base, _BPW)], idx_u)
    pltpu.sync_copy(items_hbm.at[pl.ds(base, _BPW)], idx_i)
    pltpu.sync_copy(w_hbm, w_v)

    copies = []
    for c in range(_NCHUNK):
        s = pl.ds(c * _CHUNK, _CHUNK)
        copies.append(pltpu.async_copy(u_emb_hbm.at[idx_u.at[s]], u_rows.at[s], sem))
        copies.append(pltpu.async_copy(i_emb_hbm.at[idx_i.at[s]], i_rows.at[s], sem))
    for cp in copies:
        cp.wait()

    lanes = lax.iota(jnp.int32, _D)
    w_vec = w_v[...]

    def grp(g, carry):
        au, ai = carry
        base_r = g * 16
        acc = jnp.zeros((_D,), jnp.float32)
        for r in range(16):
            u = u_rows[base_r + r, :]
            i = i_rows[base_r + r, :]
            au = au + u * u
            ai = ai + i * i
            s = jnp.sum(u * i * w_vec)
            acc = jnp.where(lanes == r, s, acc)
        out_v[pl.ds(base_r, 16)] = acc
        return (au, ai)

    zero = jnp.zeros((_D,), jnp.float32)
    au, ai = lax.fori_loop(0, _NGRP, grp, (zero, zero))
    au_v[...] = au
    ai_v[...] = ai

    pltpu.sync_copy(out_v, out_hbm.at[pl.ds(base, _BPW)])
    pltpu.sync_copy(au_v, parts_hbm.at[0, wid])
    pltpu.sync_copy(ai_v, parts_hbm.at[1, wid])


def kernel(users, items, users_ratings, items_ratings, U_emb, I_emb, W1):
    w = W1[0]
    norm = jnp.sqrt(jnp.sum(w * w))
    wn = w / jnp.maximum(norm, 1.0)
    out_flat, parts = _gmf_sc(users, items, U_emb, I_emb, wn)
    inference = out_flat.reshape(_B, 1)
    regs = _REG * (jnp.sqrt(jnp.sum(parts[0])) + jnp.sqrt(jnp.sum(parts[1])))
    return (inference, regs)
